# Initial kernel scaffold; baseline (speedup 1.0000x reference)
import jax
import jax.numpy as jnp
from jax.experimental import pallas as pl


def kernel(x, c, act):
    raise NotImplementedError("write your pallas kernel here")


# speedup vs baseline: 1.3956x; 1.3956x over previous
import functools

import jax
import jax.numpy as jnp
from jax.experimental import pallas as pl
from jax.experimental.pallas import tpu as pltpu

_MIN_CLIP = 1e-06


def _partial_kernel(x_ref, act_ref, o_ref):
    x = x_ref[0]
    a = act_ref[0]
    x2 = jnp.sum(x * x, axis=0, keepdims=True)
    ones = jnp.ones_like(x2)
    xa = jnp.concatenate([x, x2, ones], axis=0)
    m = jax.lax.dot_general(
        xa, a, (((1,), (1,)), ((), ())),
        preferred_element_type=jnp.float32)
    w = pl.program_id(1)

    @pl.when(w == 0)
    def _init():
        o_ref[0] = m

    @pl.when(w != 0)
    def _acc():
        o_ref[0] += m


def _combine_kernel(p_ref, c_ref, o_ref):
    d = c_ref.shape[0]
    m = jnp.sum(p_ref[...], axis=0)
    c = c_ref[...]
    c2 = jnp.sum(c * c, axis=0, keepdims=True)
    loss = (-2.0 * jnp.sum(c * m[:d])
            + jnp.sum(m[d:d + 1])
            + jnp.sum(c2 * m[d + 1:d + 2]))
    o_ref[0, 0] = jnp.maximum(loss, _MIN_CLIP)


@jax.jit
def kernel(x, c, act):
    n, d, wh = x.shape
    ch = c.shape[1]
    wb = wh // 2
    grid = (n, wh // wb)
    partials = pl.pallas_call(
        _partial_kernel,
        grid=grid,
        in_specs=[
            pl.BlockSpec((1, d, wb), lambda i, j: (i, 0, j)),
            pl.BlockSpec((1, ch, wb), lambda i, j: (i, 0, j)),
        ],
        out_specs=pl.BlockSpec((1, d + 2, ch), lambda i, j: (i, 0, 0)),
        out_shape=jax.ShapeDtypeStruct((n, d + 2, ch), jnp.float32),
        compiler_params=pltpu.CompilerParams(
            dimension_semantics=("parallel", "arbitrary")),
    )(x, act)

    loss = pl.pallas_call(
        _combine_kernel,
        out_specs=pl.BlockSpec(memory_space=pltpu.SMEM),
        out_shape=jax.ShapeDtypeStruct((1, 1), jnp.float32),
    )(partials, c)
    return loss[0, 0]


# speedup vs baseline: 1.5091x; 1.0814x over previous
import functools

import jax
import jax.numpy as jnp
from jax.experimental import pallas as pl
from jax.experimental.pallas import tpu as pltpu

_MIN_CLIP = 1e-06


def _partial_kernel(x_ref, act_ref, o_ref):
    x = x_ref[0]
    a = act_ref[0]
    x2 = jnp.sum(x * x, axis=0, keepdims=True)
    ones = jnp.ones_like(x2)
    xa = jnp.concatenate([x, x2, ones], axis=0)
    o_ref[0] = jax.lax.dot_general(
        xa, a, (((1,), (1,)), ((), ())),
        preferred_element_type=jnp.float32)


def _combine_kernel(p_ref, c_ref, o_ref):
    d = c_ref.shape[0]
    m = jnp.sum(p_ref[...], axis=0)
    c = c_ref[...]
    c2 = jnp.sum(c * c, axis=0, keepdims=True)
    loss = (-2.0 * jnp.sum(c * m[:d])
            + jnp.sum(m[d:d + 1])
            + jnp.sum(c2 * m[d + 1:d + 2]))
    o_ref[0, 0] = jnp.maximum(loss, _MIN_CLIP)


@jax.jit
def kernel(x, c, act):
    n, d, wh = x.shape
    ch = c.shape[1]
    partials = pl.pallas_call(
        _partial_kernel,
        grid=(n,),
        in_specs=[
            pl.BlockSpec((1, d, wh), lambda i: (i, 0, 0)),
            pl.BlockSpec((1, ch, wh), lambda i: (i, 0, 0)),
        ],
        out_specs=pl.BlockSpec((1, d + 2, ch), lambda i: (i, 0, 0)),
        out_shape=jax.ShapeDtypeStruct((n, d + 2, ch), jnp.float32),
        compiler_params=pltpu.CompilerParams(
            dimension_semantics=("parallel",)),
    )(x, act)

    loss = pl.pallas_call(
        _combine_kernel,
        out_specs=pl.BlockSpec(memory_space=pltpu.SMEM),
        out_shape=jax.ShapeDtypeStruct((1, 1), jnp.float32),
    )(partials, c)
    return loss[0, 0]


# speedup vs baseline: 1.5856x; 1.0506x over previous
import functools

import jax
import jax.numpy as jnp
from jax.experimental import pallas as pl
from jax.experimental.pallas import tpu as pltpu

_MIN_CLIP = 1e-06


def _loss_kernel(x_ref, act_ref, c_ref, o_ref, acc_ref):
    n = pl.num_programs(0)
    i = pl.program_id(0)
    d = c_ref.shape[0]
    x = x_ref[0]
    a = act_ref[0]
    x2 = jnp.sum(x * x, axis=0, keepdims=True)
    ones = jnp.ones_like(x2)
    xa = jnp.concatenate([x, x2, ones], axis=0)
    m = jax.lax.dot_general(
        xa, a, (((1,), (1,)), ((), ())),
        preferred_element_type=jnp.float32)
    c = c_ref[...]
    c2 = jnp.sum(c * c, axis=0, keepdims=True)
    part = (-2.0 * jnp.sum(c * m[:d])
            + jnp.sum(m[d:d + 1])
            + jnp.sum(c2 * m[d + 1:d + 2]))

    @pl.when(i == 0)
    def _init():
        acc_ref[0] = part

    @pl.when(i != 0)
    def _acc():
        acc_ref[0] += part

    @pl.when(i == n - 1)
    def _fin():
        o_ref[0, 0] = jnp.maximum(acc_ref[0], _MIN_CLIP)


@jax.jit
def kernel(x, c, act):
    n, d, wh = x.shape
    ch = c.shape[1]
    loss = pl.pallas_call(
        _loss_kernel,
        grid=(n,),
        in_specs=[
            pl.BlockSpec((1, d, wh), lambda i: (i, 0, 0)),
            pl.BlockSpec((1, ch, wh), lambda i: (i, 0, 0)),
            pl.BlockSpec((d, ch), lambda i: (0, 0)),
        ],
        out_specs=pl.BlockSpec(memory_space=pltpu.SMEM),
        out_shape=jax.ShapeDtypeStruct((1, 1), jnp.float32),
        scratch_shapes=[pltpu.SMEM((1,), jnp.float32)],
        compiler_params=pltpu.CompilerParams(
            dimension_semantics=("arbitrary",)),
    )(x, act, c)
    return loss[0, 0]
